# baseline (speedup 1.0000x reference)
import functools

import jax
import jax.numpy as jnp
from jax import lax
from jax.experimental import pallas as pl
from jax.experimental.pallas import tpu as pltpu
from jax.experimental.pallas import tpu_sc as plsc

B = 1024
D = 16
V = 100000
G = 8 * D

_NC = 2
_NS = 16
_NW = _NC * _NS
_B_PER_W = B // _NW

_LB = 8192


def _regroup_kernel(in_ref, out_ref):
    x = in_ref[...]
    w = x.T.reshape(_LB // 8, 8, D)
    out_ref[...] = jnp.concatenate([w[:, j, :] for j in range(8)], axis=1)


def _regroup(w_in_t):
    grid = (V + _LB - 1) // _LB
    return pl.pallas_call(
        _regroup_kernel,
        grid=(grid,),
        in_specs=[pl.BlockSpec((D, _LB), lambda i: (0, i))],
        out_specs=pl.BlockSpec((_LB // 8, G), lambda i: (i, 0)),
        out_shape=jax.ShapeDtypeStruct((grid * _LB // 8, G), jnp.float32),
        compiler_params=pltpu.CompilerParams(
            dimension_semantics=("parallel",),
        ),
    )(w_in_t)


def _gather_groups(table, idx):
    mesh = plsc.VectorSubcoreMesh(core_axis_name="c", subcore_axis_name="s")

    @functools.partial(
        pl.kernel,
        mesh=mesh,
        out_type=jax.ShapeDtypeStruct((B, G), table.dtype),
        scratch_types=[
            pltpu.VMEM((_B_PER_W,), jnp.int32),
            pltpu.VMEM((_B_PER_W, G), table.dtype),
            pltpu.SemaphoreType.DMA,
        ],
    )
    def k(table_hbm, idx_hbm, out_hbm, idx_v, rows_v, sem):
        wid = lax.axis_index("s") * _NC + lax.axis_index("c")
        base = wid * _B_PER_W
        pltpu.sync_copy(idx_hbm.at[pl.ds(base, _B_PER_W)], idx_v)
        pltpu.async_copy(table_hbm.at[idx_v], rows_v, sem).wait()
        pltpu.sync_copy(rows_v, out_hbm.at[pl.ds(base, _B_PER_W)])

    return k(table, idx)


_VB = 2048


def _matmul_kernel(w_ref, g_ref, r_ref, out_ref, emb_ref):
    @pl.when(pl.program_id(0) == 0)
    def _():
        grp = g_ref[...]
        r = r_ref[...]
        emb = jnp.zeros((B, D), jnp.float32)
        for j in range(8):
            emb = emb + jnp.where(r == j, grp[:, j * D:(j + 1) * D], 0.0)
        emb_ref[...] = emb.astype(jnp.bfloat16)

    out_ref[...] = lax.dot_general(
        w_ref[...].astype(jnp.bfloat16),
        emb_ref[...],
        dimension_numbers=(((0,), (1,)), ((), ())),
        preferred_element_type=jnp.float32,
    )


def _logits_t(w_out_t, groups, r):
    grid = (V + _VB - 1) // _VB
    return pl.pallas_call(
        _matmul_kernel,
        grid=(grid,),
        in_specs=[
            pl.BlockSpec((D, _VB), lambda i: (0, i)),
            pl.BlockSpec((B, G), lambda i: (0, 0)),
            pl.BlockSpec((B, 1), lambda i: (0, 0)),
        ],
        out_specs=pl.BlockSpec((_VB, B), lambda i: (i, 0)),
        out_shape=jax.ShapeDtypeStruct((V, B), jnp.float32),
        scratch_shapes=[pltpu.VMEM((B, D), jnp.bfloat16)],
        compiler_params=pltpu.CompilerParams(
            dimension_semantics=("arbitrary",),
        ),
    )(w_out_t, groups, r)


def kernel(x, W_in, W_out):
    idx = x.astype(jnp.int32)
    table = _regroup(W_in.T)
    groups = _gather_groups(table, idx >> 3)
    return _logits_t(W_out.T, groups, (idx & 7).reshape(B, 1)).T


# speedup vs baseline: 1.0897x; 1.0897x over previous
import functools

import jax
import jax.numpy as jnp
from jax import lax
from jax.experimental import pallas as pl
from jax.experimental.pallas import tpu as pltpu
from jax.experimental.pallas import tpu_sc as plsc

B = 1024
D = 16
V = 100000
G = 128 * D

_NC = 2
_NS = 16
_NW = _NC * _NS
_B_PER_W = B // _NW

_LB = 8192


def _regroup_kernel(in_ref, out_ref):
    x = in_ref[...]
    z = x.reshape(D, _LB // 128, 128).transpose(1, 0, 2)
    out_ref[...] = z.reshape(_LB // 128, G)


def _regroup(w_in_t):
    grid = (V + _LB - 1) // _LB
    return pl.pallas_call(
        _regroup_kernel,
        grid=(grid,),
        in_specs=[pl.BlockSpec((D, _LB), lambda i: (0, i))],
        out_specs=pl.BlockSpec((_LB // 128, G), lambda i: (i, 0)),
        out_shape=jax.ShapeDtypeStruct((grid * _LB // 128, G), jnp.float32),
        compiler_params=pltpu.CompilerParams(
            dimension_semantics=("parallel",),
        ),
    )(w_in_t)


def _gather_rows(table, idx):
    mesh = plsc.VectorSubcoreMesh(core_axis_name="c", subcore_axis_name="s")

    @functools.partial(
        pl.kernel,
        mesh=mesh,
        out_type=jax.ShapeDtypeStruct((B, G), table.dtype),
        scratch_types=[
            pltpu.VMEM((_B_PER_W,), jnp.int32),
            pltpu.VMEM((_B_PER_W, G), table.dtype),
            pltpu.SemaphoreType.DMA,
        ],
    )
    def k(table_hbm, idx_hbm, out_hbm, idx_v, rows_v, sem):
        wid = lax.axis_index("s") * _NC + lax.axis_index("c")
        base = wid * _B_PER_W
        pltpu.sync_copy(idx_hbm.at[pl.ds(base, _B_PER_W)], idx_v)
        pltpu.async_copy(table_hbm.at[idx_v], rows_v, sem).wait()
        pltpu.sync_copy(rows_v, out_hbm.at[pl.ds(base, _B_PER_W)])

    return k(table, idx)


_VB = 2048


def _matmul_kernel(w_ref, g_ref, l_ref, out_ref, emb_ref):
    @pl.when(pl.program_id(0) == 0)
    def _():
        grp = g_ref[...]
        sel = lax.broadcasted_iota(jnp.int32, (B, 128), 1) == l_ref[...]
        parts = []
        for d in range(D):
            picked = jnp.where(sel, grp[:, d * 128:(d + 1) * 128], 0.0)
            parts.append(jnp.sum(picked, axis=1, keepdims=True))
        emb_ref[...] = jnp.concatenate(parts, axis=1).astype(jnp.bfloat16)

    out_ref[...] = lax.dot_general(
        w_ref[...].astype(jnp.bfloat16),
        emb_ref[...],
        dimension_numbers=(((0,), (1,)), ((), ())),
        preferred_element_type=jnp.float32,
    )


def _logits_t(w_out_t, rows, lane):
    grid = (V + _VB - 1) // _VB
    return pl.pallas_call(
        _matmul_kernel,
        grid=(grid,),
        in_specs=[
            pl.BlockSpec((D, _VB), lambda i: (0, i)),
            pl.BlockSpec((B, G), lambda i: (0, 0)),
            pl.BlockSpec((B, 1), lambda i: (0, 0)),
        ],
        out_specs=pl.BlockSpec((_VB, B), lambda i: (i, 0)),
        out_shape=jax.ShapeDtypeStruct((V, B), jnp.float32),
        scratch_shapes=[pltpu.VMEM((B, D), jnp.bfloat16)],
        compiler_params=pltpu.CompilerParams(
            dimension_semantics=("arbitrary",),
        ),
    )(w_out_t, rows, lane)


def kernel(x, W_in, W_out):
    idx = x.astype(jnp.int32)
    table = _regroup(W_in.T)
    rows = _gather_rows(table, idx >> 7)
    return _logits_t(W_out.T, rows, (idx & 127).reshape(B, 1)).T


# speedup vs baseline: 1.0987x; 1.0082x over previous
import functools

import jax
import jax.numpy as jnp
from jax import lax
from jax.experimental import pallas as pl
from jax.experimental.pallas import tpu as pltpu
from jax.experimental.pallas import tpu_sc as plsc

B = 1024
D = 16
V = 100000
G = 128 * D

_NC = 2
_NS = 16
_NW = _NC * _NS
_B_PER_W = B // _NW

_LB = 32768


def _regroup_kernel(in_ref, out_ref):
    x = in_ref[...]
    z = x.reshape(D, _LB // 128, 128).transpose(1, 0, 2)
    out_ref[...] = z.reshape(_LB // 128, G)


def _regroup(w_in_t):
    grid = (V + _LB - 1) // _LB
    return pl.pallas_call(
        _regroup_kernel,
        grid=(grid,),
        in_specs=[pl.BlockSpec((D, _LB), lambda i: (0, i))],
        out_specs=pl.BlockSpec((_LB // 128, G), lambda i: (i, 0)),
        out_shape=jax.ShapeDtypeStruct((grid * _LB // 128, G), jnp.float32),
        compiler_params=pltpu.CompilerParams(
            dimension_semantics=("parallel",),
        ),
    )(w_in_t)


def _gather_rows(table, idx):
    mesh = plsc.VectorSubcoreMesh(core_axis_name="c", subcore_axis_name="s")

    @functools.partial(
        pl.kernel,
        mesh=mesh,
        out_type=jax.ShapeDtypeStruct((B, G), table.dtype),
        scratch_types=[
            pltpu.VMEM((_B_PER_W,), jnp.int32),
            pltpu.VMEM((_B_PER_W, G), table.dtype),
            pltpu.SemaphoreType.DMA,
        ],
    )
    def k(table_hbm, idx_hbm, out_hbm, idx_v, rows_v, sem):
        wid = lax.axis_index("s") * _NC + lax.axis_index("c")
        base = wid * _B_PER_W
        pltpu.sync_copy(idx_hbm.at[pl.ds(base, _B_PER_W)], idx_v)
        pltpu.async_copy(table_hbm.at[idx_v], rows_v, sem).wait()
        pltpu.sync_copy(rows_v, out_hbm.at[pl.ds(base, _B_PER_W)])

    return k(table, idx)


_VB = 4096


def _matmul_kernel(w_ref, g_ref, l_ref, out_ref, emb_ref):
    @pl.when(pl.program_id(0) == 0)
    def _():
        grp = g_ref[...]
        sel = lax.broadcasted_iota(jnp.int32, (B, 128), 1) == l_ref[...]
        parts = []
        for d in range(D):
            picked = jnp.where(sel, grp[:, d * 128:(d + 1) * 128], 0.0)
            parts.append(jnp.sum(picked, axis=1, keepdims=True))
        emb_ref[...] = jnp.concatenate(parts, axis=1).astype(jnp.bfloat16)

    out_ref[...] = lax.dot_general(
        w_ref[...].astype(jnp.bfloat16),
        emb_ref[...],
        dimension_numbers=(((0,), (1,)), ((), ())),
        preferred_element_type=jnp.float32,
    )


def _logits_t(w_out_t, rows, lane):
    grid = (V + _VB - 1) // _VB
    return pl.pallas_call(
        _matmul_kernel,
        grid=(grid,),
        in_specs=[
            pl.BlockSpec((D, _VB), lambda i: (0, i)),
            pl.BlockSpec((B, G), lambda i: (0, 0)),
            pl.BlockSpec((B, 1), lambda i: (0, 0)),
        ],
        out_specs=pl.BlockSpec((_VB, B), lambda i: (i, 0)),
        out_shape=jax.ShapeDtypeStruct((V, B), jnp.float32),
        scratch_shapes=[pltpu.VMEM((B, D), jnp.bfloat16)],
        compiler_params=pltpu.CompilerParams(
            dimension_semantics=("arbitrary",),
        ),
    )(w_out_t, rows, lane)


def kernel(x, W_in, W_out):
    idx = x.astype(jnp.int32)
    table = _regroup(W_in.T)
    rows = _gather_rows(table, idx >> 7)
    return _logits_t(W_out.T, rows, (idx & 127).reshape(B, 1)).T


# speedup vs baseline: 1.1403x; 1.0379x over previous
import functools

import jax
import jax.numpy as jnp
from jax import lax
from jax.experimental import pallas as pl
from jax.experimental.pallas import tpu as pltpu
from jax.experimental.pallas import tpu_sc as plsc

B = 1024
D = 16
V = 100000
G = 128 * (D // 2)

_NC = 2
_NS = 16
_NW = _NC * _NS
_B_PER_W = B // _NW

_LB = 32768


def _regroup_kernel(in_ref, out_ref):
    x = in_ref[...]
    zz = x.reshape(D, _LB // 128, 128).transpose(1, 0, 2)
    u_lo = lax.bitcast_convert_type(zz[:, :D // 2, :], jnp.uint32)
    u_hi = lax.bitcast_convert_type(zz[:, D // 2:, :], jnp.uint32)
    packed = (((u_hi + 0x8000) & jnp.uint32(0xFFFF0000))
              | ((u_lo + 0x8000) >> 16))
    out_ref[...] = lax.bitcast_convert_type(packed, jnp.int32).reshape(
        _LB // 128, G)


def _regroup(w_in_t):
    grid = (V + _LB - 1) // _LB
    return pl.pallas_call(
        _regroup_kernel,
        grid=(grid,),
        in_specs=[pl.BlockSpec((D, _LB), lambda i: (0, i))],
        out_specs=pl.BlockSpec((_LB // 128, G), lambda i: (i, 0)),
        out_shape=jax.ShapeDtypeStruct((grid * _LB // 128, G), jnp.int32),
        compiler_params=pltpu.CompilerParams(
            dimension_semantics=("parallel",),
        ),
    )(w_in_t)


def _gather_rows(table, idx):
    mesh = plsc.VectorSubcoreMesh(core_axis_name="c", subcore_axis_name="s")

    @functools.partial(
        pl.kernel,
        mesh=mesh,
        out_type=jax.ShapeDtypeStruct((B, G), table.dtype),
        scratch_types=[
            pltpu.VMEM((_B_PER_W,), jnp.int32),
            pltpu.VMEM((_B_PER_W, G), table.dtype),
            pltpu.SemaphoreType.DMA,
        ],
    )
    def k(table_hbm, idx_hbm, out_hbm, idx_v, rows_v, sem):
        wid = lax.axis_index("s") * _NC + lax.axis_index("c")
        base = wid * _B_PER_W
        pltpu.sync_copy(idx_hbm.at[pl.ds(base, _B_PER_W)], idx_v)
        pltpu.async_copy(table_hbm.at[idx_v], rows_v, sem).wait()
        pltpu.sync_copy(rows_v, out_hbm.at[pl.ds(base, _B_PER_W)])

    return k(table, idx)


_VB = 2048


def _matmul_kernel(w_ref, g_ref, l_ref, out_ref, emb_ref):
    @pl.when(pl.program_id(0) == 0)
    def _():
        u = lax.bitcast_convert_type(g_ref[...], jnp.uint32)
        f_lo = lax.bitcast_convert_type(u << 16, jnp.float32)
        f_hi = lax.bitcast_convert_type(u & jnp.uint32(0xFFFF0000),
                                        jnp.float32)
        sel = lax.broadcasted_iota(jnp.int32, (B, 128), 1) == l_ref[...]
        parts = []
        for half in (f_lo, f_hi):
            for m in range(D // 2):
                picked = jnp.where(sel, half[:, m * 128:(m + 1) * 128], 0.0)
                parts.append(jnp.sum(picked, axis=1, keepdims=True))
        emb_ref[...] = jnp.concatenate(parts, axis=1).astype(jnp.bfloat16)

    out_ref[...] = lax.dot_general(
        w_ref[...].astype(jnp.bfloat16),
        emb_ref[...],
        dimension_numbers=(((0,), (1,)), ((), ())),
        preferred_element_type=jnp.float32,
    )


def _logits_t(w_out_t, rows, lane):
    grid = (V + _VB - 1) // _VB
    return pl.pallas_call(
        _matmul_kernel,
        grid=(grid,),
        in_specs=[
            pl.BlockSpec((D, _VB), lambda i: (0, i)),
            pl.BlockSpec((B, G), lambda i: (0, 0)),
            pl.BlockSpec((B, 1), lambda i: (0, 0)),
        ],
        out_specs=pl.BlockSpec((_VB, B), lambda i: (i, 0)),
        out_shape=jax.ShapeDtypeStruct((V, B), jnp.float32),
        scratch_shapes=[pltpu.VMEM((B, D), jnp.bfloat16)],
        compiler_params=pltpu.CompilerParams(
            dimension_semantics=("arbitrary",),
        ),
    )(w_out_t, rows, lane)


def kernel(x, W_in, W_out):
    idx = x.astype(jnp.int32)
    table = _regroup(W_in.T)
    rows = _gather_rows(table, idx >> 7)
    return _logits_t(W_out.T, rows, (idx & 127).reshape(B, 1)).T
